# baseline (speedup 1.0000x reference)
import functools

import jax
import jax.numpy as jnp
from jax import lax
from jax.experimental import pallas as pl
from jax.experimental.pallas import tpu as pltpu
from jax.experimental.pallas import tpu_sc as plsc

_M = 16384
_K = 8192
_D = 64

_ROWS = 512
_GRID = _M // _ROWS
_CHUNK = 4096

_LOSS_SCALE = 1.25 / (_M * _D)


def _dist_argmin_body(x_ref, cb_ref, x2_ref, e2_ref, idx_ref, loss_ref,
                      cbs_ref):
    i = pl.program_id(0)

    @pl.when(i == 0)
    def _init():
        cbs_ref[...] = (cb_ref[...] * jnp.float32(-2.0)).astype(jnp.bfloat16)
        loss_ref[...] = jnp.zeros((1, 1), jnp.float32)

    x = x_ref[...].astype(jnp.bfloat16)
    mm2 = lax.dot_general(x, cbs_ref[...],
                          (((1,), (1,)), ((), ())),
                          preferred_element_type=jnp.float32)
    d2 = (x2_ref[...] + mm2) + e2_ref[...]
    acc_r = jnp.full((_ROWS, 1), jnp.inf, jnp.float32)
    acc_e = jnp.full((_ROWS, 1), jnp.inf, jnp.float32)
    acc_i = jnp.zeros((_ROWS, 1), jnp.int32)
    for c in range(_K // _CHUNK):
        dc = d2[:, c * _CHUNK:(c + 1) * _CHUNK]
        vc = jnp.min(dc, axis=1, keepdims=True)
        iota = lax.broadcasted_iota(jnp.int32, (_ROWS, _CHUNK), 1)
        ic = jnp.min(jnp.where(dc == vc, iota + c * _CHUNK, _K),
                     axis=1, keepdims=True)
        take = vc < acc_r
        acc_i = jnp.where(take, ic, acc_i)
        acc_e = jnp.where(take, vc, acc_e)
        acc_r = jnp.where(take, vc.astype(jnp.bfloat16).astype(jnp.float32),
                          acc_r)
    idx_ref[0, :, :] = acc_i
    loss_ref[...] += (jnp.sum(acc_e) * jnp.float32(_LOSS_SCALE)).reshape(1, 1)


def _dist_argmin(flat, codebook, x2, e2):
    return pl.pallas_call(
        _dist_argmin_body,
        grid=(_GRID,),
        in_specs=[
            pl.BlockSpec((_ROWS, _D), lambda i: (i, 0)),
            pl.BlockSpec((_K, _D), lambda i: (0, 0)),
            pl.BlockSpec((_ROWS, 1), lambda i: (i, 0)),
            pl.BlockSpec((1, _K), lambda i: (0, 0)),
        ],
        out_specs=[
            pl.BlockSpec((1, _ROWS, 1), lambda i: (i, 0, 0)),
            pl.BlockSpec((1, 1), lambda i: (0, 0)),
        ],
        out_shape=[
            jax.ShapeDtypeStruct((_GRID, _ROWS, 1), jnp.int32),
            jax.ShapeDtypeStruct((1, 1), jnp.float32),
        ],
        scratch_shapes=[
            pltpu.VMEM((_K, _D), jnp.bfloat16),
        ],
    )(flat, codebook, x2, e2)


@functools.cache
def _make_sc_gather():
    info = plsc.get_sparse_core_info()
    nw = info.num_cores * info.num_subcores
    b_per_w = _M // nw
    mesh = plsc.VectorSubcoreMesh(core_axis_name="c", subcore_axis_name="s")

    @functools.partial(
        pl.kernel,
        mesh=mesh,
        out_type=jax.ShapeDtypeStruct((_M, _D), jnp.float32),
        scratch_types=[
            pltpu.VMEM((b_per_w,), jnp.int32),
            pltpu.VMEM((b_per_w, _D), jnp.float32),
            pltpu.SemaphoreType.DMA,
        ],
        compiler_params=pltpu.CompilerParams(use_tc_tiling_on_sc=False),
    )
    def gather_kernel(table_hbm, idx_hbm, out_hbm, idx_v, rows_v, sem):
        wid = lax.axis_index("s") * info.num_cores + lax.axis_index("c")
        base = wid * b_per_w
        pltpu.sync_copy(idx_hbm.at[pl.ds(base, b_per_w)], idx_v)
        pltpu.async_copy(table_hbm.at[idx_v], rows_v, sem).wait()
        pltpu.sync_copy(rows_v, out_hbm.at[pl.ds(base, b_per_w)])

    return gather_kernel


def kernel(z, codebook):
    B, N, D = z.shape
    flat = z.reshape(B * N, D)
    x2 = jnp.sum(z * z, axis=2).reshape(B * N, 1)
    e2 = jnp.sum(codebook * codebook, axis=1).reshape(1, _K)
    idx3, loss = _dist_argmin(flat, codebook, x2, e2)
    idx_flat = idx3.reshape(B * N)
    zq = _make_sc_gather()(codebook, idx_flat)
    out = zq.reshape(B, N, D)
    return out, loss[0, 0], idx_flat.reshape(B, N)


# speedup vs baseline: 1.0359x; 1.0359x over previous
import functools

import jax
import jax.numpy as jnp
from jax import lax
from jax.experimental import pallas as pl
from jax.experimental.pallas import tpu as pltpu
from jax.experimental.pallas import tpu_sc as plsc

_M = 16384
_K = 8192
_D = 64

_ROWS = 1024
_GRID = _M // _ROWS
_CHUNK = 4096

_LOSS_SCALE = 1.25 / (_M * _D)


def _dist_argmin_body(x_ref, cb_ref, x2_ref, e2_ref, idx_ref, loss_ref,
                      cbs_ref):
    i = pl.program_id(0)

    @pl.when(i == 0)
    def _init():
        cbs_ref[...] = cb_ref[...] * jnp.float32(-2.0)
        loss_ref[...] = jnp.zeros((1, 1), jnp.float32)

    x = x_ref[...]
    mm2 = lax.dot_general(x, cbs_ref[...],
                          (((1,), (1,)), ((), ())),
                          preferred_element_type=jnp.float32)
    d2 = (x2_ref[...] + mm2) + e2_ref[...]
    acc_r = jnp.full((_ROWS, 1), jnp.inf, jnp.float32)
    acc_e = jnp.full((_ROWS, 1), jnp.inf, jnp.float32)
    acc_i = jnp.zeros((_ROWS, 1), jnp.int32)
    for c in range(_K // _CHUNK):
        dc = d2[:, c * _CHUNK:(c + 1) * _CHUNK]
        vc = jnp.min(dc, axis=1, keepdims=True)
        iota = lax.broadcasted_iota(jnp.int32, (_ROWS, _CHUNK), 1)
        ic = jnp.min(jnp.where(dc == vc, iota + c * _CHUNK, _K),
                     axis=1, keepdims=True)
        take = vc < acc_r
        acc_i = jnp.where(take, ic, acc_i)
        acc_e = jnp.where(take, vc, acc_e)
        acc_r = jnp.where(take, vc.astype(jnp.bfloat16).astype(jnp.float32),
                          acc_r)
    idx_ref[0, :, :] = acc_i
    loss_ref[...] += (jnp.sum(acc_e) * jnp.float32(_LOSS_SCALE)).reshape(1, 1)


def _dist_argmin(flat, codebook, x2, e2):
    return pl.pallas_call(
        _dist_argmin_body,
        grid=(_GRID,),
        in_specs=[
            pl.BlockSpec((_ROWS, _D), lambda i: (i, 0)),
            pl.BlockSpec((_K, _D), lambda i: (0, 0)),
            pl.BlockSpec((_ROWS, 1), lambda i: (i, 0)),
            pl.BlockSpec((1, _K), lambda i: (0, 0)),
        ],
        out_specs=[
            pl.BlockSpec((1, _ROWS, 1), lambda i: (i, 0, 0)),
            pl.BlockSpec((1, 1), lambda i: (0, 0)),
        ],
        out_shape=[
            jax.ShapeDtypeStruct((_GRID, _ROWS, 1), jnp.int32),
            jax.ShapeDtypeStruct((1, 1), jnp.float32),
        ],
        scratch_shapes=[
            pltpu.VMEM((_K, _D), jnp.float32),
        ],
    )(flat, codebook, x2, e2)


@functools.cache
def _make_sc_gather():
    info = plsc.get_sparse_core_info()
    nw = info.num_cores * info.num_subcores
    b_per_w = _M // nw
    mesh = plsc.VectorSubcoreMesh(core_axis_name="c", subcore_axis_name="s")

    @functools.partial(
        pl.kernel,
        mesh=mesh,
        out_type=jax.ShapeDtypeStruct((_M, _D), jnp.float32),
        scratch_types=[
            pltpu.VMEM((b_per_w,), jnp.int32),
            pltpu.VMEM((b_per_w, _D), jnp.float32),
            pltpu.SemaphoreType.DMA,
        ],
        compiler_params=pltpu.CompilerParams(use_tc_tiling_on_sc=False),
    )
    def gather_kernel(table_hbm, idx_hbm, out_hbm, idx_v, rows_v, sem):
        wid = lax.axis_index("s") * info.num_cores + lax.axis_index("c")
        base = wid * b_per_w
        pltpu.sync_copy(idx_hbm.at[pl.ds(base, b_per_w)], idx_v)
        pltpu.async_copy(table_hbm.at[idx_v], rows_v, sem).wait()
        pltpu.sync_copy(rows_v, out_hbm.at[pl.ds(base, b_per_w)])

    return gather_kernel


def kernel(z, codebook):
    B, N, D = z.shape
    flat = z.reshape(B * N, D)
    x2 = jnp.sum(z * z, axis=2).reshape(B * N, 1)
    e2 = jnp.sum(codebook * codebook, axis=1).reshape(1, _K)
    idx3, loss = _dist_argmin(flat, codebook, x2, e2)
    idx_flat = idx3.reshape(B * N)
    zq = _make_sc_gather()(codebook, idx_flat)
    out = zq.reshape(B, N, D)
    return out, loss[0, 0], idx_flat.reshape(B, N)
